# Initial kernel scaffold; baseline (speedup 1.0000x reference)
import jax
import jax.numpy as jnp
from jax import lax
from jax.experimental import pallas as pl
from jax.experimental.pallas import tpu as pltpu
from jax.experimental.pallas import tpu_sc as plsc

DIM = 64
NC, NS = 2, 16
NW = NC * NS
CHUNK = 128


def _gather_body(cpw):
    def body(x_hbm, w_hbm, out_hbm, idx_v, rows_v, gsem):
        wid = lax.axis_index("s") * NC + lax.axis_index("c")
        pltpu.sync_copy(x_hbm.at[pl.ds(wid * cpw, cpw)], idx_v)

        def step(g, carry):
            pltpu.async_copy(w_hbm.at[idx_v.at[g]], rows_v, gsem).wait()
            pltpu.sync_copy(rows_v, out_hbm.at[pl.ds((wid * cpw + g) * CHUNK, CHUNK)])
            return carry

        lax.fori_loop(0, cpw, step, 0)
    return body


def kernel(x, weight):
    B, H = x.shape
    N = B * H
    assert N % (NW * CHUNK) == 0
    cpw = N // (NW * CHUNK)
    xf = x.reshape(N // CHUNK, CHUNK).astype(jnp.int32)
    mesh = plsc.VectorSubcoreMesh(
        core_axis_name="c", subcore_axis_name="s", num_cores=NC, num_subcores=NS
    )
    out = pl.kernel(
        _gather_body(cpw),
        out_type=jax.ShapeDtypeStruct((N, DIM), jnp.float32),
        mesh=mesh,
        scratch_types=[
            pltpu.VMEM((cpw, CHUNK), jnp.int32),
            pltpu.VMEM((CHUNK, DIM), jnp.float32),
            pltpu.SemaphoreType.DMA,
        ],
    )(xf, weight)
    return out.reshape(B, H, DIM)


# speedup vs baseline: 1.6853x; 1.6853x over previous
import jax
import jax.numpy as jnp
from jax import lax
from jax.experimental import pallas as pl
from jax.experimental.pallas import tpu as pltpu
from jax.experimental.pallas import tpu_sc as plsc

DIM = 64
NC, NS = 2, 16
NW = NC * NS
CHUNK = 128


def _gather_body(cpw):
    def body(x_hbm, w_hbm, out_hbm, idx_v, rows_v, gsem):
        wid = lax.axis_index("s") * NC + lax.axis_index("c")
        pltpu.sync_copy(x_hbm.at[pl.ds(wid * cpw, cpw)], idx_v)

        def step(g, carry):
            pltpu.async_copy(w_hbm.at[idx_v.at[g]], rows_v, gsem).wait()
            pltpu.sync_copy(rows_v, out_hbm.at[pl.ds((wid * cpw + g) * CHUNK, CHUNK)])
            return carry

        lax.fori_loop(0, cpw, step, 0)
    return body


def kernel(x, weight):
    B, H = x.shape
    N = B * H
    assert N % (NW * CHUNK) == 0
    cpw = N // (NW * CHUNK)
    xf = x.reshape(N // CHUNK, CHUNK).astype(jnp.int32)
    mesh = plsc.VectorSubcoreMesh(
        core_axis_name="c", subcore_axis_name="s", num_cores=NC, num_subcores=NS
    )
    out = pl.kernel(
        _gather_body(cpw),
        out_type=jax.ShapeDtypeStruct((N, DIM), jnp.float32),
        mesh=mesh,
        compiler_params=pltpu.CompilerParams(use_tc_tiling_on_sc=False),
        scratch_types=[
            pltpu.VMEM((cpw, CHUNK), jnp.int32),
            pltpu.VMEM((CHUNK, DIM), jnp.float32),
            pltpu.SemaphoreType.DMA,
        ],
    )(xf, weight)
    return out.reshape(B, H, DIM)


# speedup vs baseline: 1.8740x; 1.1120x over previous
import jax
import jax.numpy as jnp
from jax import lax
from jax.experimental import pallas as pl
from jax.experimental.pallas import tpu as pltpu
from jax.experimental.pallas import tpu_sc as plsc

DIM = 64
NC, NS = 2, 16
NW = NC * NS
CHUNK = 128
K = 5


def _gather_body(cpw):
    ngroups = cpw // K
    npairs = ngroups // 2
    grp_rows = K * CHUNK

    def body(x_hbm, w_hbm, out_hbm, idx_v, buf_a, buf_b, gsem_a, gsem_b,
             wsem_a, wsem_b):
        wid = lax.axis_index("s") * NC + lax.axis_index("c")
        base_chunk = wid * cpw
        base_row = base_chunk * CHUNK
        pltpu.sync_copy(x_hbm.at[pl.ds(base_chunk, cpw)], idx_v)

        def fire(buf, gsem, k):
            for j in range(K):
                pltpu.async_copy(
                    w_hbm.at[idx_v.at[k * K + j]],
                    buf.at[pl.ds(j * CHUNK, CHUNK)], gsem)

        def drain_gather(buf, gsem):
            pltpu.make_async_copy(out_hbm.at[pl.ds(0, grp_rows)], buf,
                                  gsem).wait()

        def start_write(buf, wsem, k):
            pltpu.async_copy(
                buf, out_hbm.at[pl.ds(base_row + k * grp_rows, grp_rows)],
                wsem)

        def wait_write(buf, wsem):
            pltpu.make_async_copy(
                buf, out_hbm.at[pl.ds(base_row, grp_rows)], wsem).wait()

        fire(buf_a, gsem_a, 0)

        def pair(p, carry):
            ka = 2 * p
            kb = 2 * p + 1
            drain_gather(buf_a, gsem_a)

            @pl.when(p > 0)
            def _():
                wait_write(buf_b, wsem_b)

            fire(buf_b, gsem_b, kb)
            start_write(buf_a, wsem_a, ka)

            drain_gather(buf_b, gsem_b)

            @pl.when(p < npairs - 1)
            def _():
                wait_write(buf_a, wsem_a)
                fire(buf_a, gsem_a, ka + 2)

            start_write(buf_b, wsem_b, kb)
            return carry

        lax.fori_loop(0, npairs, pair, 0)
        wait_write(buf_a, wsem_a)
        wait_write(buf_b, wsem_b)
    return body


def kernel(x, weight):
    B, H = x.shape
    N = B * H
    assert N % (NW * CHUNK) == 0
    cpw = N // (NW * CHUNK)
    assert cpw % (2 * K) == 0
    xf = x.reshape(N // CHUNK, CHUNK).astype(jnp.int32)
    mesh = plsc.VectorSubcoreMesh(
        core_axis_name="c", subcore_axis_name="s", num_cores=NC, num_subcores=NS
    )
    out = pl.kernel(
        _gather_body(cpw),
        out_type=jax.ShapeDtypeStruct((N, DIM), jnp.float32),
        mesh=mesh,
        compiler_params=pltpu.CompilerParams(use_tc_tiling_on_sc=False),
        scratch_types=[
            pltpu.VMEM((cpw, CHUNK), jnp.int32),
            pltpu.VMEM((K * CHUNK, DIM), jnp.float32),
            pltpu.VMEM((K * CHUNK, DIM), jnp.float32),
            pltpu.SemaphoreType.DMA,
            pltpu.SemaphoreType.DMA,
            pltpu.SemaphoreType.DMA,
            pltpu.SemaphoreType.DMA,
        ],
    )(xf, weight)
    return out.reshape(B, H, DIM)
